# baseline (speedup 1.0000x reference)
import functools

import jax
import jax.numpy as jnp
from jax import lax
from jax.experimental import pallas as pl
from jax.experimental.pallas import tpu as pltpu
from jax.experimental.pallas import tpu_sc as plsc

N = 10000
D = 128
E = 320000

NC = 2
NS = 16

CHUNK = 128
CPP = 40
NBUF = 2

FAST_CORE = 0
NPH_FAST = 2
NPH_SLOW = 2
FAST_CHUNKS = NS * NPH_FAST * CPP
TOTAL_CHUNKS = FAST_CHUNKS + NS * NPH_SLOW * CPP
E_PAD = TOTAL_CHUNKS * CHUNK

N_ACC = 10240
ROWS_PER_TILE = N_ACC // NS
LAST_START = 15 * ROWS_PER_TILE
LAST_ROWS = N - LAST_START


def _sc_partials(x, src_chunks, dst_chunks):
  mesh = plsc.VectorSubcoreMesh(core_axis_name="c", subcore_axis_name="s")

  @functools.partial(
      pl.kernel,
      out_type=jax.ShapeDtypeStruct((NC, N, D), jnp.float32),
      mesh=mesh,
      scratch_types=[
          pltpu.VMEM((CPP, CHUNK), jnp.int32),
          pltpu.VMEM((CPP, CHUNK), jnp.int32),
          pltpu.VMEM((NBUF, CHUNK, D), jnp.float32),
          pltpu.VMEM_SHARED((N_ACC, D), jnp.float32),
      ] + [pltpu.SemaphoreType.DMA] * (2 * NBUF),
  )
  def k(x_hbm, srcc_hbm, dstc_hbm, out_hbm, sidx_v, didx_v, rows_v, acc_sh,
        *sems):
    gsems = sems[:NBUF]
    ssems = sems[NBUF:]
    cid = lax.axis_index("c")
    sid = lax.axis_index("s")

    is_fast = cid == FAST_CORE
    nph = jnp.where(is_fast, NPH_FAST, NPH_SLOW)
    tile_base = jnp.where(is_fast, sid * (NPH_FAST * CPP),
                          FAST_CHUNKS + sid * (NPH_SLOW * CPP))

    zbuf = rows_v.at[0]

    @pl.loop(0, CHUNK)
    def _(r):
      row = zbuf.at[r]
      for c in range(D // 16):
        row[pl.ds(c * 16, 16)] = jnp.zeros((16,), jnp.float32)

    acc_start = sid * ROWS_PER_TILE
    for z in range(ROWS_PER_TILE // CHUNK):
      pltpu.sync_copy(zbuf, acc_sh.at[pl.ds(acc_start + z * CHUNK, CHUNK)])

    plsc.subcore_barrier()

    @pl.loop(0, nph)
    def _(p):
      pbase = tile_base + p * CPP
      pltpu.sync_copy(srcc_hbm.at[pl.ds(pbase, CPP)], sidx_v)
      pltpu.sync_copy(dstc_hbm.at[pl.ds(pbase, CPP)], didx_v)

      for b in range(NBUF):
        pltpu.async_copy(x_hbm.at[sidx_v.at[b]], rows_v.at[b], gsems[b])

      @pl.loop(0, CPP - NBUF, step=NBUF)
      def _(j0):
        for b in range(NBUF):
          j = j0 + b
          pltpu.make_async_copy(x_hbm.at[sidx_v.at[j]], rows_v.at[b],
                                gsems[b]).wait()
          sd = pltpu.async_copy(rows_v.at[b], acc_sh.at[didx_v.at[j]],
                                ssems[b], add=True)
          sd.wait()
          pltpu.async_copy(x_hbm.at[sidx_v.at[j + NBUF]], rows_v.at[b],
                           gsems[b])

      tail = []
      for b in range(NBUF):
        j = CPP - NBUF + b
        pltpu.make_async_copy(x_hbm.at[sidx_v.at[j]], rows_v.at[b],
                              gsems[b]).wait()
        tail.append(pltpu.async_copy(rows_v.at[b], acc_sh.at[didx_v.at[j]],
                                     ssems[b], add=True))
      for sd in tail:
        sd.wait()

    plsc.subcore_barrier()

    @pl.when(sid < NS - 1)
    def _():
      pltpu.sync_copy(acc_sh.at[pl.ds(acc_start, ROWS_PER_TILE)],
                      out_hbm.at[cid, pl.ds(acc_start, ROWS_PER_TILE)])

    @pl.when(sid == NS - 1)
    def _():
      pltpu.sync_copy(acc_sh.at[pl.ds(LAST_START, LAST_ROWS)],
                      out_hbm.at[cid, pl.ds(LAST_START, LAST_ROWS)])

  return k(x, src_chunks, dst_chunks)


def _add_kernel(a_ref, b_ref, o_ref):
  o_ref[...] = a_ref[...] + b_ref[...]


def _combine(p0, p1):
  blk = 1000
  return pl.pallas_call(
      _add_kernel,
      out_shape=jax.ShapeDtypeStruct((N, D), jnp.float32),
      grid=(N // blk,),
      in_specs=[
          pl.BlockSpec((blk, D), lambda i: (i, 0)),
          pl.BlockSpec((blk, D), lambda i: (i, 0)),
      ],
      out_specs=pl.BlockSpec((blk, D), lambda i: (i, 0)),
  )(p0, p1)


@jax.jit
def kernel(x, edge_index):
  src = edge_index[0]
  dst = edge_index[1]
  pad = E_PAD - E
  pad_dst = N + (jnp.arange(pad, dtype=jnp.int32) % (N_ACC - N))
  src_p = jnp.concatenate([src, jnp.zeros((pad,), jnp.int32)])
  dst_p = jnp.concatenate([dst, pad_dst])
  src_chunks = src_p.reshape(TOTAL_CHUNKS, CHUNK)
  dst_chunks = dst_p.reshape(TOTAL_CHUNKS, CHUNK)
  partials = _sc_partials(x, src_chunks, dst_chunks)
  return _combine(partials[0], partials[1])


# speedup vs baseline: 1.1585x; 1.1585x over previous
import functools

import jax
import jax.numpy as jnp
from jax import lax
from jax.experimental import pallas as pl
from jax.experimental.pallas import tpu as pltpu
from jax.experimental.pallas import tpu_sc as plsc

N = 10000
D = 128
E = 320000

NC = 2
NS = 16

CHUNK = 128
CPP = 40
NBUF = 2

FAST_CORE = 0
NPH_FAST = 2
NPH_SLOW = 2
FAST_CHUNKS = NS * NPH_FAST * CPP
TOTAL_CHUNKS = FAST_CHUNKS + NS * NPH_SLOW * CPP
E_PAD = TOTAL_CHUNKS * CHUNK

N_ACC = 10240
ROWS_PER_TILE = N_ACC // NS
LAST_START = 15 * ROWS_PER_TILE
LAST_ROWS = N - LAST_START


def _sc_partials(x, src_chunks, dst_chunks):
  mesh = plsc.VectorSubcoreMesh(core_axis_name="c", subcore_axis_name="s")

  @functools.partial(
      pl.kernel,
      out_type=jax.ShapeDtypeStruct((NC, N, D), jnp.float32),
      mesh=mesh,
      scratch_types=[
          pltpu.VMEM((CPP, CHUNK), jnp.int32),
          pltpu.VMEM((CPP, CHUNK), jnp.int32),
          pltpu.VMEM((NBUF, CHUNK, D), jnp.float32),
          pltpu.VMEM_SHARED((N_ACC, D), jnp.float32),
      ] + [pltpu.SemaphoreType.DMA] * (2 * NBUF),
  )
  def k(x_hbm, srcc_hbm, dstc_hbm, out_hbm, sidx_v, didx_v, rows_v, acc_sh,
        *sems):
    gsems = sems[:NBUF]
    ssems = sems[NBUF:]
    cid = lax.axis_index("c")
    sid = lax.axis_index("s")

    nph = NPH_FAST

    zbuf = rows_v.at[0]

    @pl.loop(0, CHUNK)
    def _(r):
      row = zbuf.at[r]
      for c in range(D // 16):
        row[pl.ds(c * 16, 16)] = jnp.zeros((16,), jnp.float32)

    acc_start = sid * ROWS_PER_TILE
    for z in range(ROWS_PER_TILE // CHUNK):
      pltpu.sync_copy(zbuf, acc_sh.at[pl.ds(acc_start + z * CHUNK, CHUNK)])

    plsc.subcore_barrier()

    @pl.loop(0, nph)
    def _(p):
      pbase = (((p * NS) + sid) * NC + cid) * CPP
      pltpu.sync_copy(srcc_hbm.at[pl.ds(pbase, CPP)], sidx_v)
      pltpu.sync_copy(dstc_hbm.at[pl.ds(pbase, CPP)], didx_v)

      for b in range(NBUF):
        pltpu.async_copy(x_hbm.at[sidx_v.at[b]], rows_v.at[b], gsems[b])

      @pl.loop(0, CPP - NBUF, step=NBUF)
      def _(j0):
        for b in range(NBUF):
          j = j0 + b
          pltpu.make_async_copy(x_hbm.at[sidx_v.at[j]], rows_v.at[b],
                                gsems[b]).wait()
          sd = pltpu.async_copy(rows_v.at[b], acc_sh.at[didx_v.at[j]],
                                ssems[b], add=True)
          sd.wait()
          pltpu.async_copy(x_hbm.at[sidx_v.at[j + NBUF]], rows_v.at[b],
                           gsems[b])

      tail = []
      for b in range(NBUF):
        j = CPP - NBUF + b
        pltpu.make_async_copy(x_hbm.at[sidx_v.at[j]], rows_v.at[b],
                              gsems[b]).wait()
        tail.append(pltpu.async_copy(rows_v.at[b], acc_sh.at[didx_v.at[j]],
                                     ssems[b], add=True))
      for sd in tail:
        sd.wait()

    plsc.subcore_barrier()

    @pl.when(sid < NS - 1)
    def _():
      pltpu.sync_copy(acc_sh.at[pl.ds(acc_start, ROWS_PER_TILE)],
                      out_hbm.at[cid, pl.ds(acc_start, ROWS_PER_TILE)])

    @pl.when(sid == NS - 1)
    def _():
      pltpu.sync_copy(acc_sh.at[pl.ds(LAST_START, LAST_ROWS)],
                      out_hbm.at[cid, pl.ds(LAST_START, LAST_ROWS)])

  return k(x, src_chunks, dst_chunks)


def _add_kernel(a_ref, b_ref, o_ref):
  o_ref[...] = a_ref[...] + b_ref[...]


def _combine(p0, p1):
  blk = 1000
  return pl.pallas_call(
      _add_kernel,
      out_shape=jax.ShapeDtypeStruct((N, D), jnp.float32),
      grid=(N // blk,),
      in_specs=[
          pl.BlockSpec((blk, D), lambda i: (i, 0)),
          pl.BlockSpec((blk, D), lambda i: (i, 0)),
      ],
      out_specs=pl.BlockSpec((blk, D), lambda i: (i, 0)),
  )(p0, p1)


@jax.jit
def kernel(x, edge_index):
  src = edge_index[0]
  dst = edge_index[1]
  pad = E_PAD - E
  pad_dst = N + (jnp.arange(pad, dtype=jnp.int32) % (N_ACC - N))
  src_p = jnp.concatenate([src, jnp.zeros((pad,), jnp.int32)])
  dst_p = jnp.concatenate([dst, pad_dst])
  src_chunks = src_p.reshape(TOTAL_CHUNKS, CHUNK)
  dst_chunks = dst_p.reshape(TOTAL_CHUNKS, CHUNK)
  partials = _sc_partials(x, src_chunks, dst_chunks)
  return _combine(partials[0], partials[1])


# speedup vs baseline: 1.2607x; 1.0882x over previous
import functools

import jax
import jax.numpy as jnp
from jax import lax
from jax.experimental import pallas as pl
from jax.experimental.pallas import tpu as pltpu
from jax.experimental.pallas import tpu_sc as plsc

N = 10000
D = 128
E = 320000

NC = 2
NS = 16

CHUNK = 128
NBUF = 2

CPP0 = 48
CPP1 = 32
CPPS = CPP0 + CPP1
NPH = 2
TOTAL_CHUNKS = NS * NPH * CPPS
E_PAD = TOTAL_CHUNKS * CHUNK

N_ACC = 10240
ROWS_PER_TILE = N_ACC // NS
LAST_START = 15 * ROWS_PER_TILE
LAST_ROWS = N - LAST_START


def _sc_partials(x, src_chunks, dst_chunks):
  mesh = plsc.VectorSubcoreMesh(core_axis_name="c", subcore_axis_name="s")

  @functools.partial(
      pl.kernel,
      out_type=jax.ShapeDtypeStruct((NC, N, D), jnp.float32),
      mesh=mesh,
      scratch_types=[
          pltpu.VMEM((CPP0, CHUNK), jnp.int32),
          pltpu.VMEM((CPP0, CHUNK), jnp.int32),
          pltpu.VMEM((NBUF, CHUNK, D), jnp.float32),
          pltpu.VMEM_SHARED((N_ACC, D), jnp.float32),
      ] + [pltpu.SemaphoreType.DMA] * (2 * NBUF),
  )
  def k(x_hbm, srcc_hbm, dstc_hbm, out_hbm, sidx_v, didx_v, rows_v, acc_sh,
        *sems):
    gsems = sems[:NBUF]
    ssems = sems[NBUF:]
    cid = lax.axis_index("c")
    sid = lax.axis_index("s")

    cpp = jnp.where(cid == 0, CPP0, CPP1)

    zbuf = rows_v.at[0]

    @pl.loop(0, CHUNK)
    def _(r):
      row = zbuf.at[r]
      for c in range(D // 16):
        row[pl.ds(c * 16, 16)] = jnp.zeros((16,), jnp.float32)

    acc_start = sid * ROWS_PER_TILE
    for z in range(ROWS_PER_TILE // CHUNK):
      pltpu.sync_copy(zbuf, acc_sh.at[pl.ds(acc_start + z * CHUNK, CHUNK)])

    plsc.subcore_barrier()

    @pl.loop(0, NPH)
    def _(p):
      pbase = ((p * NS) + sid) * CPPS + jnp.where(cid == 0, 0, CPP0)

      @pl.when(cid == 0)
      def _():
        pltpu.sync_copy(srcc_hbm.at[pl.ds(pbase, CPP0)], sidx_v)
        pltpu.sync_copy(dstc_hbm.at[pl.ds(pbase, CPP0)], didx_v)

      @pl.when(cid != 0)
      def _():
        pltpu.sync_copy(srcc_hbm.at[pl.ds(pbase, CPP1)],
                        sidx_v.at[pl.ds(0, CPP1)])
        pltpu.sync_copy(dstc_hbm.at[pl.ds(pbase, CPP1)],
                        didx_v.at[pl.ds(0, CPP1)])

      for b in range(NBUF):
        pltpu.async_copy(x_hbm.at[sidx_v.at[b]], rows_v.at[b], gsems[b])

      @pl.loop(0, cpp - NBUF, step=NBUF)
      def _(j0):
        for b in range(NBUF):
          j = j0 + b
          pltpu.make_async_copy(x_hbm.at[sidx_v.at[j]], rows_v.at[b],
                                gsems[b]).wait()
          sd = pltpu.async_copy(rows_v.at[b], acc_sh.at[didx_v.at[j]],
                                ssems[b], add=True)
          sd.wait()
          pltpu.async_copy(x_hbm.at[sidx_v.at[j + NBUF]], rows_v.at[b],
                           gsems[b])

      tail = []
      for b in range(NBUF):
        j = cpp - NBUF + b
        pltpu.make_async_copy(x_hbm.at[sidx_v.at[j]], rows_v.at[b],
                              gsems[b]).wait()
        tail.append(pltpu.async_copy(rows_v.at[b], acc_sh.at[didx_v.at[j]],
                                     ssems[b], add=True))
      for sd in tail:
        sd.wait()

    plsc.subcore_barrier()

    @pl.when(sid < NS - 1)
    def _():
      pltpu.sync_copy(acc_sh.at[pl.ds(acc_start, ROWS_PER_TILE)],
                      out_hbm.at[cid, pl.ds(acc_start, ROWS_PER_TILE)])

    @pl.when(sid == NS - 1)
    def _():
      pltpu.sync_copy(acc_sh.at[pl.ds(LAST_START, LAST_ROWS)],
                      out_hbm.at[cid, pl.ds(LAST_START, LAST_ROWS)])

  return k(x, src_chunks, dst_chunks)


def _add_kernel(a_ref, b_ref, o_ref):
  o_ref[...] = a_ref[...] + b_ref[...]


def _combine(p0, p1):
  blk = 1000
  return pl.pallas_call(
      _add_kernel,
      out_shape=jax.ShapeDtypeStruct((N, D), jnp.float32),
      grid=(N // blk,),
      in_specs=[
          pl.BlockSpec((blk, D), lambda i: (i, 0)),
          pl.BlockSpec((blk, D), lambda i: (i, 0)),
      ],
      out_specs=pl.BlockSpec((blk, D), lambda i: (i, 0)),
  )(p0, p1)


@jax.jit
def kernel(x, edge_index):
  src = edge_index[0]
  dst = edge_index[1]
  pad = E_PAD - E
  pad_dst = N + (jnp.arange(pad, dtype=jnp.int32) % (N_ACC - N))
  src_p = jnp.concatenate([src, jnp.zeros((pad,), jnp.int32)])
  dst_p = jnp.concatenate([dst, pad_dst])
  src_chunks = src_p.reshape(TOTAL_CHUNKS, CHUNK)
  dst_chunks = dst_p.reshape(TOTAL_CHUNKS, CHUNK)
  partials = _sc_partials(x, src_chunks, dst_chunks)
  return _combine(partials[0], partials[1])
